# baseline (speedup 1.0000x reference)
import functools

import jax
import jax.numpy as jnp
from jax import lax
from jax.experimental import pallas as pl
from jax.experimental.pallas import tpu as pltpu
from jax.experimental.pallas import tpu_sc as plsc

_B = 16384
_D = 20
_L = 16


def kernel(lengths, table):
    lengths = lengths.astype(jnp.int32)
    info = plsc.get_sparse_core_info()
    nw = info.num_cores * info.num_subcores
    b_per_w = _B // nw
    mesh = plsc.VectorSubcoreMesh(core_axis_name="c", subcore_axis_name="s")

    @functools.partial(
        pl.kernel,
        mesh=mesh,
        out_type=jax.ShapeDtypeStruct((_B, _D), jnp.float32),
        scratch_types=[
            pltpu.VMEM((b_per_w,), jnp.int32),
            pltpu.VMEM((10, _D), jnp.float32),
            pltpu.VMEM((b_per_w, _D), jnp.float32),
        ],
        compiler_params=pltpu.CompilerParams(needs_layout_passes=False),
    )
    def sc_kernel(lengths_hbm, table_hbm, out_hbm, len_v, table_v, out_v):
        wid = lax.axis_index("s") * info.num_cores + lax.axis_index("c")
        base = wid * b_per_w
        pltpu.sync_copy(lengths_hbm.at[pl.ds(base, b_per_w)], len_v)
        pltpu.sync_copy(table_hbm, table_v)

        lane = lax.iota(jnp.int32, _L)
        zero = lane * 0
        cols = [zero + c for c in range(_D)]

        @plsc.parallel_loop(0, b_per_w // _L, unroll=4)
        def body(c):
            lv = len_v[pl.ds(c * _L, _L)]
            bv = jnp.minimum(lv, 5)
            for t in (8, 16, 32, 64):
                bv = bv + jnp.where(lv >= t, 1, 0).astype(jnp.int32)
            rows16 = lane + c * _L
            for col in range(_D):
                vals = plsc.load_gather(table_v, [bv, cols[col]])
                plsc.store_scatter(out_v, [rows16, cols[col]], vals)

        pltpu.sync_copy(out_v, out_hbm.at[pl.ds(base, b_per_w)])

    return sc_kernel(lengths, table)


# speedup vs baseline: 1.1107x; 1.1107x over previous
import functools

import jax
import jax.numpy as jnp
from jax import lax
from jax.experimental import pallas as pl
from jax.experimental.pallas import tpu as pltpu
from jax.experimental.pallas import tpu_sc as plsc

_B = 16384
_D = 20
_L = 16


def kernel(lengths, table):
    lengths = lengths.astype(jnp.int32)
    info = plsc.get_sparse_core_info()
    nw = info.num_cores * info.num_subcores
    b_per_w = _B // nw
    mesh = plsc.VectorSubcoreMesh(core_axis_name="c", subcore_axis_name="s")

    @functools.partial(
        pl.kernel,
        mesh=mesh,
        out_type=jax.ShapeDtypeStruct((_B, _D), jnp.float32),
        scratch_types=[
            pltpu.VMEM((b_per_w,), jnp.int32),
            pltpu.VMEM((10, _D), jnp.float32),
            pltpu.VMEM((b_per_w, _D), jnp.float32),
        ],
        compiler_params=pltpu.CompilerParams(needs_layout_passes=False),
    )
    def sc_kernel(lengths_hbm, table_hbm, out_hbm, len_v, table_v, out_v):
        wid = lax.axis_index("s") * info.num_cores + lax.axis_index("c")
        base = wid * b_per_w
        pltpu.sync_copy(lengths_hbm.at[pl.ds(base, b_per_w)], len_v)
        pltpu.sync_copy(table_hbm, table_v)

        lane = lax.iota(jnp.int32, _L)
        zero = lane * 0
        cols = [zero + c for c in range(_D)]

        @plsc.parallel_loop(0, b_per_w // _L, unroll=1)
        def body(c):
            lv = len_v[pl.ds(c * _L, _L)]
            bv = jnp.minimum(lv, 5)
            for t in (8, 16, 32, 64):
                bv = bv + jnp.where(lv >= t, 1, 0).astype(jnp.int32)
            rows16 = lane + c * _L
            for col in range(_D):
                vals = plsc.load_gather(table_v, [bv, cols[col]])
                plsc.store_scatter(out_v, [rows16, cols[col]], vals)

        pltpu.sync_copy(out_v, out_hbm.at[pl.ds(base, b_per_w)])

    return sc_kernel(lengths, table)
